# baseline (device time: 59508 ns/iter reference)
import jax
import jax.numpy as jnp
from jax import lax
from jax.experimental import pallas as pl
from jax.experimental.pallas import tpu as pltpu

B = 16
H = 16
D = 64
BS = 16
NP_LOC = 128
NB = 128
NKEY = NP_LOC * BS
BH = B * H
HD = H * D
SCALE = D ** -0.5


def kernel(Q, K, V, bt, lens):
    lens2d = lens.reshape(B, 1)
    q_rows = Q.reshape(BH, D)
    km = K.reshape(NKEY, HD)
    vm = V.reshape(NKEY, HD)

    def body(q_ref, k_ref, v_ref, bt_ref, lens_ref, out_ref,
             comm_ref, send_sem, recv_sem):
        my_x = lax.axis_index("x")
        my_y = lax.axis_index("y")
        my_z = lax.axis_index("z")

        bt3 = bt_ref[:, :][:, :, None]
        pids = lax.broadcasted_iota(jnp.int32, (B, NB, NP_LOC), 2) \
            + my_x * NP_LOC
        jidx = lax.broadcasted_iota(jnp.int32, (B, NB, 1), 1)
        valid = jidx < lens_ref[:, :][:, None, :]
        hit = jnp.logical_and(bt3 == pids, valid)
        cnt = jnp.sum(jnp.where(hit, 1.0, 0.0), axis=1)

        en = jnp.where(
            lax.broadcasted_iota(jnp.int32, (BH, B), 0) // H
            == lax.broadcasted_iota(jnp.int32, (BH, B), 1),
            1.0, 0.0).astype(jnp.bfloat16)
        ek = jnp.where(
            lax.broadcasted_iota(jnp.int32, (NP_LOC, NKEY), 0)
            == lax.broadcasted_iota(jnp.int32, (NP_LOC, NKEY), 1) // BS,
            1.0, 0.0).astype(jnp.bfloat16)
        cb = lax.dot_general(en, cnt.astype(jnp.bfloat16),
                             (((1,), (0,)), ((), ())),
                             preferred_element_type=jnp.float32)
        c_nk = lax.dot_general(cb.astype(jnp.bfloat16), ek,
                               (((1,), (0,)), ((), ())),
                               preferred_element_type=jnp.float32)

        q_tile = jnp.concatenate([q_ref[:, :]] * H, axis=1)
        col_h = lax.broadcasted_iota(jnp.int32, (BH, HD), 1) // D
        row_h = lax.broadcasted_iota(jnp.int32, (BH, HD), 0) % H
        w = jnp.where(col_h == row_h, q_tile, 0.0).astype(jnp.bfloat16)

        s = lax.dot_general(
            w, k_ref[:, :].astype(jnp.bfloat16),
            (((1,), (1,)), ((), ())),
            preferred_element_type=jnp.float32,
        ) * SCALE
        s = jnp.where(c_nk > 0.0, s, -1e30)
        m = jnp.max(s, axis=1, keepdims=True)
        p = c_nk * jnp.exp(s - m)
        l = jnp.sum(p, axis=1, keepdims=True)
        o_big = lax.dot_general(
            p.astype(jnp.bfloat16), v_ref[:, :].astype(jnp.bfloat16),
            (((1,), (0,)), ((), ())),
            preferred_element_type=jnp.float32,
        )
        o_sel = jnp.where(col_h == row_h, o_big, 0.0)
        acc = o_sel[:, 0:D]
        for hh in range(1, H):
            acc = acc + o_sel[:, hh * D:(hh + 1) * D]

        comm_ref[0, :, 0:D] = acc
        comm_ref[0, :, D:D + 1] = m
        comm_ref[0, :, D + 1:D + 2] = l

        peer = (1 - my_x, my_y, my_z)
        barrier_sem = pltpu.get_barrier_semaphore()
        pl.semaphore_signal(barrier_sem, inc=1, device_id=peer,
                            device_id_type=pl.DeviceIdType.MESH)
        pl.semaphore_wait(barrier_sem, 1)

        rdma = pltpu.make_async_remote_copy(
            src_ref=comm_ref.at[0],
            dst_ref=comm_ref.at[1],
            send_sem=send_sem,
            recv_sem=recv_sem,
            device_id=peer,
            device_id_type=pl.DeviceIdType.MESH,
        )
        rdma.start()
        rdma.wait()

        acc1 = comm_ref[0, :, 0:D]
        m1 = comm_ref[0, :, D:D + 1]
        l1 = comm_ref[0, :, D + 1:D + 2]
        acc2 = comm_ref[1, :, 0:D]
        m2 = comm_ref[1, :, D:D + 1]
        l2 = comm_ref[1, :, D + 1:D + 2]
        m_new = jnp.maximum(m1, m2)
        a1 = jnp.exp(m1 - m_new)
        a2 = jnp.exp(m2 - m_new)
        l_tot = l1 * a1 + l2 * a2
        out = (acc1 * a1 + acc2 * a2) / l_tot
        out_ref[:, 0, :, :] = out.reshape(B, H, D)

    return pl.pallas_call(
        body,
        out_shape=jax.ShapeDtypeStruct((B, 1, H, D), jnp.float32),
        in_specs=[
            pl.BlockSpec(memory_space=pltpu.VMEM),
            pl.BlockSpec(memory_space=pltpu.VMEM),
            pl.BlockSpec(memory_space=pltpu.VMEM),
            pl.BlockSpec(memory_space=pltpu.VMEM),
            pl.BlockSpec(memory_space=pltpu.VMEM),
        ],
        out_specs=pl.BlockSpec(memory_space=pltpu.VMEM),
        scratch_shapes=[
            pltpu.VMEM((2, BH, 128), jnp.float32),
            pltpu.SemaphoreType.DMA,
            pltpu.SemaphoreType.DMA,
        ],
        compiler_params=pltpu.CompilerParams(collective_id=0),
    )(q_rows, km, vm, bt, lens2d)
